# baseline (speedup 1.0000x reference)
import functools

import jax
import jax.numpy as jnp
from jax import lax
from jax.experimental import pallas as pl
from jax.experimental.pallas import tpu as pltpu
from jax.experimental.pallas import tpu_sc as plsc

N = 524288
F = 128
P = 16

N_SC = 131072
N_TC = N - N_SC

L = 16
NW = 32
ROWS_PER_W = N_SC // NW
R = 256
CH = R * F
NCHUNK = ROWS_PER_W // R
GROUPS = R // L

BLOCK = 49152

_mesh = plsc.VectorSubcoreMesh(core_axis_name="c", subcore_axis_name="s")


def _take16(vec, idx):
    dnums = lax.GatherDimensionNumbers(
        offset_dims=(), collapsed_slice_dims=(0,), start_index_map=(0,))
    return lax.gather(vec, idx[:, None], dnums, (1,),
                      mode=lax.GatherScatterMode.PROMISE_IN_BOUNDS)


@functools.partial(
    pl.kernel,
    out_type=jax.ShapeDtypeStruct((N_SC,), jnp.float32),
    mesh=_mesh,
    compiler_params=pltpu.CompilerParams(needs_layout_passes=False),
    scratch_types=[
        pltpu.VMEM((CH,), jnp.float32),
        pltpu.VMEM((CH,), jnp.float32),
        pltpu.VMEM((F,), jnp.float32),
        pltpu.VMEM((P,), jnp.int32),
        pltpu.VMEM((ROWS_PER_W,), jnp.float32),
        pltpu.SemaphoreType.DMA,
        pltpu.SemaphoreType.DMA,
    ],
)
def _sc_match(x_hbm, th_hbm, pat_hbm, out_hbm, buf0, buf1, th_v, pat_v, out_v,
              sem0, sem1):
    wid = lax.axis_index("s") * 2 + lax.axis_index("c")
    base = (N_TC + wid * ROWS_PER_W) * F

    pltpu.sync_copy(th_hbm, th_v)
    pltpu.sync_copy(pat_hbm, pat_v)

    lane = lax.broadcasted_iota(jnp.int32, (L,), 0)
    pat_vec = pat_v[...]
    th_sel = plsc.load_gather(th_v, [pat_vec])
    pat_b = []
    th_b = []
    for p in range(P):
        sel = jnp.full((L,), p, jnp.int32)
        pat_b.append(_take16(pat_vec, sel))
        th_b.append(_take16(th_sel, sel))

    def start(c, buf, sem):
        pltpu.async_copy(x_hbm.at[pl.ds(base + c * CH, CH)], buf, sem)

    def wait(c, buf, sem):
        pltpu.make_async_copy(
            x_hbm.at[pl.ds(base + c * CH, CH)], buf, sem).wait()

    def compute(buf, c):
        def gbody(g, carry):
            rowbase = (g * L + lane) * F
            acc = None
            for p in range(P):
                vals = plsc.load_gather(buf, [rowbase + pat_b[p]])
                ok = vals > th_b[p]
                acc = ok if acc is None else (acc & ok)
            out_v[pl.ds(c * R + g * L, L)] = jnp.where(
                acc, jnp.ones((L,), jnp.float32), jnp.zeros((L,), jnp.float32))
            return carry

        lax.fori_loop(0, GROUPS, gbody, 0)

    start(0, buf0, sem0)

    def outer(i, carry):
        c0 = 2 * i
        start(c0 + 1, buf1, sem1)
        wait(c0, buf0, sem0)
        compute(buf0, c0)

        @pl.when(c0 + 2 < NCHUNK)
        def _():
            start(c0 + 2, buf0, sem0)

        wait(c0 + 1, buf1, sem1)
        compute(buf1, c0 + 1)
        return carry

    lax.fori_loop(0, NCHUNK // 2, outer, 0)
    pltpu.sync_copy(out_v, out_hbm.at[pl.ds(wid * ROWS_PER_W, ROWS_PER_W)])


def _tc_match_block(x_ref, th_ref, pat_ref, o_ref):
    pat = pat_ref[...]
    col = lax.broadcasted_iota(jnp.int32, (P, F), 1)
    mask = (pat[:, None] == col).any(axis=0)
    x = x_ref[...]
    th = th_ref[...]
    fail = ((x <= th[None, :]) & mask[None, :]).astype(jnp.float32)
    cnt = lax.dot_general(
        jnp.ones((1, F), jnp.float32), fail,
        dimension_numbers=(((1,), (1,)), ((), ())),
        preferred_element_type=jnp.float32,
    )
    o_ref[...] = (cnt == 0.0).astype(jnp.float32)[None]


@jax.jit
def kernel(inputs, thresholds, pat_index):
    out_sc = _sc_match(inputs.reshape(N * F), thresholds, pat_index)
    out_tc = pl.pallas_call(
        _tc_match_block,
        grid=(N_TC // BLOCK,),
        in_specs=[
            pl.BlockSpec((BLOCK, F), lambda i: (i, 0)),
            pl.BlockSpec((F,), lambda i: (0,)),
            pl.BlockSpec((P,), lambda i: (0,)),
        ],
        out_specs=pl.BlockSpec((1, 1, BLOCK), lambda i: (i, 0, 0)),
        out_shape=jax.ShapeDtypeStruct((N_TC // BLOCK, 1, BLOCK), jnp.float32),
    )(inputs, thresholds, pat_index)
    return jnp.concatenate(
        [out_tc.reshape(N_TC), out_sc]).astype(jnp.bool_)


# speedup vs baseline: 1.0275x; 1.0275x over previous
import functools

import jax
import jax.numpy as jnp
from jax import lax
from jax.experimental import pallas as pl
from jax.experimental.pallas import tpu as pltpu
from jax.experimental.pallas import tpu_sc as plsc

N = 524288
F = 128
P = 16

N_SC = 32768
N_TC = N - N_SC

L = 16
NW = 32
ROWS_PER_W = N_SC // NW
R = 256
CH = R * F
NCHUNK = ROWS_PER_W // R
GROUPS = R // L

BLOCK = 32768

_mesh = plsc.VectorSubcoreMesh(core_axis_name="c", subcore_axis_name="s")


def _take16(vec, idx):
    dnums = lax.GatherDimensionNumbers(
        offset_dims=(), collapsed_slice_dims=(0,), start_index_map=(0,))
    return lax.gather(vec, idx[:, None], dnums, (1,),
                      mode=lax.GatherScatterMode.PROMISE_IN_BOUNDS)


@functools.partial(
    pl.kernel,
    out_type=jax.ShapeDtypeStruct((N_SC,), jnp.float32),
    mesh=_mesh,
    compiler_params=pltpu.CompilerParams(needs_layout_passes=False),
    scratch_types=[
        pltpu.VMEM((CH,), jnp.float32),
        pltpu.VMEM((CH,), jnp.float32),
        pltpu.VMEM((F,), jnp.float32),
        pltpu.VMEM((P,), jnp.int32),
        pltpu.VMEM((ROWS_PER_W,), jnp.float32),
        pltpu.SemaphoreType.DMA,
        pltpu.SemaphoreType.DMA,
    ],
)
def _sc_match(x_hbm, th_hbm, pat_hbm, out_hbm, buf0, buf1, th_v, pat_v, out_v,
              sem0, sem1):
    wid = lax.axis_index("s") * 2 + lax.axis_index("c")
    base = (N_TC + wid * ROWS_PER_W) * F

    pltpu.sync_copy(th_hbm, th_v)
    pltpu.sync_copy(pat_hbm, pat_v)

    lane = lax.broadcasted_iota(jnp.int32, (L,), 0)
    pat_vec = pat_v[...]
    th_sel = plsc.load_gather(th_v, [pat_vec])
    pat_b = []
    th_b = []
    for p in range(P):
        sel = jnp.full((L,), p, jnp.int32)
        pat_b.append(_take16(pat_vec, sel))
        th_b.append(_take16(th_sel, sel))

    def start(c, buf, sem):
        pltpu.async_copy(x_hbm.at[pl.ds(base + c * CH, CH)], buf, sem)

    def wait(c, buf, sem):
        pltpu.make_async_copy(
            x_hbm.at[pl.ds(base + c * CH, CH)], buf, sem).wait()

    def compute(buf, c):
        def gbody(g, carry):
            rowbase = (g * L + lane) * F
            acc = None
            for p in range(P):
                vals = plsc.load_gather(buf, [rowbase + pat_b[p]])
                ok = vals > th_b[p]
                acc = ok if acc is None else (acc & ok)
            out_v[pl.ds(c * R + g * L, L)] = jnp.where(
                acc, jnp.ones((L,), jnp.float32), jnp.zeros((L,), jnp.float32))
            return carry

        lax.fori_loop(0, GROUPS, gbody, 0)

    start(0, buf0, sem0)

    def outer(i, carry):
        c0 = 2 * i
        start(c0 + 1, buf1, sem1)
        wait(c0, buf0, sem0)
        compute(buf0, c0)

        @pl.when(c0 + 2 < NCHUNK)
        def _():
            start(c0 + 2, buf0, sem0)

        wait(c0 + 1, buf1, sem1)
        compute(buf1, c0 + 1)
        return carry

    lax.fori_loop(0, NCHUNK // 2, outer, 0)
    pltpu.sync_copy(out_v, out_hbm.at[pl.ds(wid * ROWS_PER_W, ROWS_PER_W)])


def _tc_match_block(x_ref, th_ref, pat_ref, o_ref):
    pat = pat_ref[...]
    col = lax.broadcasted_iota(jnp.int32, (P, F), 1)
    mask = (pat[:, None] == col).any(axis=0)
    x = x_ref[...]
    th = th_ref[...]
    fail = ((x <= th[None, :]) & mask[None, :]).astype(jnp.float32)
    cnt = lax.dot_general(
        jnp.ones((1, F), jnp.float32), fail,
        dimension_numbers=(((1,), (1,)), ((), ())),
        preferred_element_type=jnp.float32,
    )
    o_ref[...] = (cnt == 0.0).astype(jnp.float32)[None]


@jax.jit
def kernel(inputs, thresholds, pat_index):
    out_sc = _sc_match(inputs.reshape(N * F), thresholds, pat_index)
    out_tc = pl.pallas_call(
        _tc_match_block,
        grid=(N_TC // BLOCK,),
        in_specs=[
            pl.BlockSpec((BLOCK, F), lambda i: (i, 0)),
            pl.BlockSpec((F,), lambda i: (0,)),
            pl.BlockSpec((P,), lambda i: (0,)),
        ],
        out_specs=pl.BlockSpec((1, 1, BLOCK), lambda i: (i, 0, 0)),
        out_shape=jax.ShapeDtypeStruct((N_TC // BLOCK, 1, BLOCK), jnp.float32),
    )(inputs, thresholds, pat_index)
    return jnp.concatenate(
        [out_tc.reshape(N_TC), out_sc]).astype(jnp.bool_)
